# baseline (speedup 1.0000x reference)
import jax
import jax.numpy as jnp
from jax.experimental import pallas as pl
from jax.experimental.pallas import tpu as pltpu
from jax.experimental.pallas import tpu_sc as plsc

NB = 16
NF = 4096
NC = 64
OUT_W = 7
LANES = 16
CF = 256
NCHAIN = 8
CLEN = NC // NCHAIN


def _sc_powerset(x2d, map_flat):
    mesh = plsc.VectorSubcoreMesh(core_axis_name="c", subcore_axis_name="s")

    @pl.kernel(
        out_type=jax.ShapeDtypeStruct((OUT_W, NB, NF), jnp.float32),
        mesh=mesh,
        scratch_types=[
            pltpu.VMEM((NC * OUT_W,), jnp.float32),
            pltpu.SemaphoreType.DMA,
        ],
        compiler_params=pltpu.CompilerParams(
            needs_layout_passes=False, skip_device_barrier=True
        ),
    )
    def k(x_hbm, map_hbm, out_hbm, map_v, sem):
        pltpu.async_copy(map_hbm, map_v, sem).wait()

        iota = jax.lax.iota(jnp.int32, LANES)

        def body(in_v, out_v):
            def emit_group(g, half):
                base = g * (2 * LANES) + half * LANES
                sl = pl.ds(base, LANES)
                v = [in_v[c, sl] for c in range(NC)]
                cmax = []
                for kk in range(NCHAIN):
                    t = v[kk * CLEN:(kk + 1) * CLEN]
                    while len(t) > 1:
                        t = [jnp.maximum(a, b) for a, b in zip(t[::2], t[1::2])]
                    cmax.append(t[0])
                t = cmax
                while len(t) > 1:
                    t = [jnp.maximum(a, b) for a, b in zip(t[::2], t[1::2])]
                big = t[0]
                cid = jnp.full((LANES,), (NCHAIN - 1) * CLEN, jnp.int32)
                for kk in range(NCHAIN - 2, -1, -1):
                    cid = jnp.where(cmax[kk] == big, kk * CLEN, cid)
                fbase = iota + base
                tw = jnp.full((LANES,), CLEN - 1, jnp.int32)
                for t_ in range(CLEN - 2, -1, -1):
                    val = plsc.load_gather(in_v, [cid + t_, fbase])
                    tw = jnp.where(val == big, t_, tw)
                bidx = cid + tw
                for j in range(OUT_W):
                    vals = plsc.load_gather(map_v, [bidx + j * NC])
                    out_v[j, 0, sl] = vals

            @pl.loop(0, CF // (2 * LANES))
            def _(g):
                emit_group(g, 0)
                emit_group(g, 1)

        ncol = NF // CF
        pltpu.emit_pipeline(
            body,
            grid=(NB * ncol,),
            in_specs=[
                pl.BlockSpec((NC, CF), lambda i: (i // ncol, i % ncol)),
            ],
            out_specs=[
                pl.BlockSpec((OUT_W, 1, CF), lambda i: (0, i // ncol, i % ncol)),
            ],
            core_axis_name=("c", "s"),
            dimension_semantics=(pltpu.PARALLEL,),
        )(x_hbm, out_hbm)

    return k(x2d, map_flat)


@jax.jit
def kernel(powerset, mapping):
    x2d = powerset.transpose(0, 2, 1).reshape(NB * NC, NF)
    map_flat = mapping.T.reshape(-1)
    out = _sc_powerset(x2d, map_flat)
    return out.transpose(1, 2, 0)


# speedup vs baseline: 1.1023x; 1.1023x over previous
import jax
import jax.numpy as jnp
from jax.experimental import pallas as pl
from jax.experimental.pallas import tpu as pltpu
from jax.experimental.pallas import tpu_sc as plsc

NB = 16
NF = 4096
NC = 64
OUT_W = 7
LANES = 16
CF = 256
NCHAIN = 8
CLEN = NC // NCHAIN


def _sc_powerset(x2d, map_flat):
    mesh = plsc.VectorSubcoreMesh(core_axis_name="c", subcore_axis_name="s")

    @pl.kernel(
        out_type=jax.ShapeDtypeStruct((OUT_W, NB, NF), jnp.float32),
        mesh=mesh,
        scratch_types=[
            pltpu.VMEM((NC * OUT_W,), jnp.float32),
            pltpu.SemaphoreType.DMA,
        ],
        compiler_params=pltpu.CompilerParams(
            needs_layout_passes=False, skip_device_barrier=True
        ),
    )
    def k(x_hbm, map_hbm, out_hbm, map_v, sem):
        pltpu.async_copy(map_hbm, map_v, sem).wait()

        iota = jax.lax.iota(jnp.int32, LANES)

        def body(in_v, out_v):
            def emit_group(g, half):
                base = g * (2 * LANES) + half * LANES
                sl = pl.ds(base, LANES)
                bests, bidxs = [], []
                for kk in range(NCHAIN):
                    c0 = kk * CLEN
                    bv = in_v[c0, sl]
                    bi = jnp.full((LANES,), c0, jnp.int32)
                    for c in range(c0 + 1, c0 + CLEN):
                        v = in_v[c, sl]
                        m = v > bv
                        bv = jnp.where(m, v, bv)
                        bi = jnp.where(m, c, bi)
                    bests.append(bv)
                    bidxs.append(bi)
                best, bidx = bests[0], bidxs[0]
                for kk in range(1, NCHAIN):
                    m = bests[kk] > best
                    best = jnp.where(m, bests[kk], best)
                    bidx = jnp.where(m, bidxs[kk], bidx)
                for j in range(OUT_W):
                    vals = plsc.load_gather(map_v, [bidx + j * NC])
                    out_v[j, 0, sl] = vals

            @pl.loop(0, CF // (2 * LANES))
            def _(g):
                emit_group(g, 0)
                emit_group(g, 1)

        ncol = NF // CF
        pltpu.emit_pipeline(
            body,
            grid=(NB * ncol,),
            in_specs=[
                pl.BlockSpec((NC, CF), lambda i: (i // ncol, i % ncol)),
            ],
            out_specs=[
                pl.BlockSpec((OUT_W, 1, CF), lambda i: (0, i // ncol, i % ncol)),
            ],
            core_axis_name=("c", "s"),
            dimension_semantics=(pltpu.PARALLEL,),
        )(x_hbm, out_hbm)

    return k(x2d, map_flat)


@jax.jit
def kernel(powerset, mapping):
    x2d = powerset.transpose(0, 2, 1).reshape(NB * NC, NF)
    map_flat = mapping.T.reshape(-1)
    out = _sc_powerset(x2d, map_flat)
    return out.transpose(1, 2, 0)


# speedup vs baseline: 1.1190x; 1.0152x over previous
import jax
import jax.numpy as jnp
from jax.experimental import pallas as pl
from jax.experimental.pallas import tpu as pltpu
from jax.experimental.pallas import tpu_sc as plsc

NB = 16
NF = 4096
NC = 64
OUT_W = 7
LANES = 16
CF = 256
NCHAIN = 8
CLEN = NC // NCHAIN


def _sc_powerset(x2d, map_flat):
    mesh = plsc.VectorSubcoreMesh(core_axis_name="c", subcore_axis_name="s")

    @pl.kernel(
        out_type=jax.ShapeDtypeStruct((OUT_W, NB, NF), jnp.float32),
        mesh=mesh,
        scratch_types=[
            pltpu.VMEM((NC * OUT_W,), jnp.float32),
            pltpu.SemaphoreType.DMA,
        ],
        compiler_params=pltpu.CompilerParams(
            needs_layout_passes=False, skip_device_barrier=True
        ),
    )
    def k(x_hbm, map_hbm, out_hbm, map_v, sem):
        pltpu.async_copy(map_hbm, map_v, sem).wait()

        iota = jax.lax.iota(jnp.int32, LANES)

        def body(in_v, out_v):
            def emit_group(g, half):
                base = g * LANES
                sl = pl.ds(base, LANES)
                vals = [in_v[c, sl] for c in range(NC)]
                idxs = [jnp.full((LANES,), c, jnp.int32) for c in range(NC)]
                while len(vals) > 1:
                    nv, ni = [], []
                    for a in range(0, len(vals), 2):
                        m = vals[a + 1] > vals[a]
                        nv.append(jnp.where(m, vals[a + 1], vals[a]))
                        ni.append(jnp.where(m, idxs[a + 1], idxs[a]))
                    vals, idxs = nv, ni
                best, bidx = vals[0], idxs[0]
                for j in range(OUT_W):
                    vals = plsc.load_gather(map_v, [bidx + j * NC])
                    out_v[j, 0, sl] = vals

            @pl.loop(0, CF // LANES)
            def _(g):
                emit_group(g, 0)

        ncol = NF // CF
        pltpu.emit_pipeline(
            body,
            grid=(NB * ncol,),
            in_specs=[
                pl.BlockSpec((NC, CF), lambda i: (i // ncol, i % ncol)),
            ],
            out_specs=[
                pl.BlockSpec((OUT_W, 1, CF), lambda i: (0, i // ncol, i % ncol)),
            ],
            core_axis_name=("c", "s"),
            dimension_semantics=(pltpu.PARALLEL,),
        )(x_hbm, out_hbm)

    return k(x2d, map_flat)


@jax.jit
def kernel(powerset, mapping):
    x2d = powerset.transpose(0, 2, 1).reshape(NB * NC, NF)
    map_flat = mapping.T.reshape(-1)
    out = _sc_powerset(x2d, map_flat)
    return out.transpose(1, 2, 0)


# speedup vs baseline: 1.1581x; 1.0350x over previous
import jax
import jax.numpy as jnp
from jax.experimental import pallas as pl
from jax.experimental.pallas import tpu as pltpu
from jax.experimental.pallas import tpu_sc as plsc

NB = 16
NF = 4096
NC = 64
OUT_W = 7
LANES = 16
CF = 256
NCHAIN = 8
CLEN = NC // NCHAIN


def _sc_powerset(x2d, map_flat):
    mesh = plsc.VectorSubcoreMesh(core_axis_name="c", subcore_axis_name="s")

    @pl.kernel(
        out_type=jax.ShapeDtypeStruct((OUT_W, NB, NF), jnp.float32),
        mesh=mesh,
        scratch_types=[
            pltpu.VMEM((NC * OUT_W,), jnp.float32),
            pltpu.SemaphoreType.DMA,
        ],
        compiler_params=pltpu.CompilerParams(
            needs_layout_passes=False, skip_device_barrier=True
        ),
    )
    def k(x_hbm, map_hbm, out_hbm, map_v, sem):
        pltpu.async_copy(map_hbm, map_v, sem).wait()

        iota = jax.lax.iota(jnp.int32, LANES)

        def body(in_v, out_v):
            def emit_group(g, half):
                base = g * LANES
                sl = pl.ds(base, LANES)
                bests, bidxs = [], []
                for kk in range(NCHAIN):
                    c0 = kk * CLEN
                    bv = in_v[c0, sl]
                    bi = jnp.full((LANES,), c0, jnp.int32)
                    for c in range(c0 + 1, c0 + CLEN):
                        v = in_v[c, sl]
                        m = v > bv
                        bv = jnp.where(m, v, bv)
                        bi = jnp.where(m, c, bi)
                    bests.append(bv)
                    bidxs.append(bi)
                best, bidx = bests[0], bidxs[0]
                for kk in range(1, NCHAIN):
                    m = bests[kk] > best
                    best = jnp.where(m, bests[kk], best)
                    bidx = jnp.where(m, bidxs[kk], bidx)
                for j in range(OUT_W):
                    vals = plsc.load_gather(map_v, [bidx + j * NC])
                    out_v[j, 0, sl] = vals

            @pl.loop(0, CF // LANES)
            def _(g):
                emit_group(g, 0)

        ncol = NF // CF
        pltpu.emit_pipeline(
            body,
            grid=(NB * ncol,),
            in_specs=[
                pl.BlockSpec((NC, CF), lambda i: (i // ncol, i % ncol)),
            ],
            out_specs=[
                pl.BlockSpec((OUT_W, 1, CF), lambda i: (0, i // ncol, i % ncol)),
            ],
            core_axis_name=("c", "s"),
            dimension_semantics=(pltpu.PARALLEL,),
        )(x_hbm, out_hbm)

    return k(x2d, map_flat)


@jax.jit
def kernel(powerset, mapping):
    x2d = powerset.transpose(0, 2, 1).reshape(NB * NC, NF)
    map_flat = mapping.T.reshape(-1)
    out = _sc_powerset(x2d, map_flat)
    return out.transpose(1, 2, 0)
